# speedup vs baseline: 4.9931x; 4.9931x over previous
import functools

import jax
import jax.numpy as jnp
from jax.experimental import pallas as pl

N_ATOMS = 500000
D_FEAT = 64
EMBED_DIM = 32
NUM_GROUPS = 19
MAX_Z = 120

BLOCK = 5000


def _body(z_ref, gm_ref, feat_ref, table_ref, out_ref):
    out_ref[:, :D_FEAT] = feat_ref[...]

    z = z_ref[...]
    gm = gm_ref[...]

    gcols = jax.lax.broadcasted_iota(jnp.int32, (MAX_Z, NUM_GROUPS), 1)
    ohg = (gm == gcols).astype(jnp.float32)
    fused = jnp.dot(ohg, table_ref[...],
                    preferred_element_type=jnp.float32)

    zcols = jax.lax.broadcasted_iota(jnp.int32, (BLOCK, MAX_Z), 1)
    ohz = (z == zcols).astype(jnp.float32)
    out_ref[:, D_FEAT:] = jnp.dot(ohz, fused,
                                  preferred_element_type=jnp.float32)


@functools.partial(jax.jit, static_argnames=())
def kernel(per_atom_property_tensor, atomic_numbers, table, group_map):
    n = per_atom_property_tensor.shape[0]
    grid = n // BLOCK
    z2d = atomic_numbers.astype(jnp.int32).reshape(n, 1)
    gm2d = group_map.astype(jnp.int32).reshape(MAX_Z, 1)
    return pl.pallas_call(
        _body,
        grid=(grid,),
        in_specs=[
            pl.BlockSpec((BLOCK, 1), lambda i: (i, 0)),
            pl.BlockSpec((MAX_Z, 1), lambda i: (0, 0)),
            pl.BlockSpec((BLOCK, D_FEAT), lambda i: (i, 0)),
            pl.BlockSpec((NUM_GROUPS, EMBED_DIM), lambda i: (0, 0)),
        ],
        out_specs=pl.BlockSpec((BLOCK, D_FEAT + EMBED_DIM), lambda i: (i, 0)),
        out_shape=jax.ShapeDtypeStruct((n, D_FEAT + EMBED_DIM), jnp.float32),
    )(z2d, gm2d, per_atom_property_tensor, table)


# speedup vs baseline: 6.8515x; 1.3722x over previous
import functools

import jax
import jax.numpy as jnp
from jax.experimental import pallas as pl

N_ATOMS = 500000
D_FEAT = 64
EMBED_DIM = 32
NUM_GROUPS = 19
MAX_Z = 120

BLOCK = 5000
ZSUB = 8
ZL = BLOCK // ZSUB


def _body(z_ref, gm_ref, feat_ref, table_ref, out_ref):
    out_ref[:, :D_FEAT] = feat_ref[...]

    z = z_ref[0, 0, :][:, None]
    gm = gm_ref[...]

    gcols = jax.lax.broadcasted_iota(jnp.int32, (MAX_Z, NUM_GROUPS), 1)
    ohg = (gm == gcols).astype(jnp.float32)
    fused = jnp.dot(ohg, table_ref[...],
                    preferred_element_type=jnp.float32)

    zcols = jax.lax.broadcasted_iota(jnp.int32, (BLOCK, MAX_Z), 1)
    ohz = (z == zcols).astype(jnp.float32)
    out_ref[:, D_FEAT:] = jnp.dot(ohz, fused,
                                  preferred_element_type=jnp.float32)


@functools.partial(jax.jit, static_argnames=())
def kernel(per_atom_property_tensor, atomic_numbers, table, group_map):
    n = per_atom_property_tensor.shape[0]
    grid = n // BLOCK
    z2d = atomic_numbers.astype(jnp.int32).reshape(grid, 1, BLOCK)
    gm2d = group_map.astype(jnp.int32).reshape(MAX_Z, 1)
    return pl.pallas_call(
        _body,
        grid=(grid,),
        in_specs=[
            pl.BlockSpec((1, 1, BLOCK), lambda i: (i, 0, 0)),
            pl.BlockSpec((MAX_Z, 1), lambda i: (0, 0)),
            pl.BlockSpec((BLOCK, D_FEAT), lambda i: (i, 0)),
            pl.BlockSpec((NUM_GROUPS, EMBED_DIM), lambda i: (0, 0)),
        ],
        out_specs=pl.BlockSpec((BLOCK, D_FEAT + EMBED_DIM), lambda i: (i, 0)),
        out_shape=jax.ShapeDtypeStruct((n, D_FEAT + EMBED_DIM), jnp.float32),
    )(z2d, gm2d, per_atom_property_tensor, table)


# speedup vs baseline: 7.3463x; 1.0722x over previous
import functools

import jax
import jax.numpy as jnp
from jax.experimental import pallas as pl

N_ATOMS = 500000
D_FEAT = 64
EMBED_DIM = 32
NUM_GROUPS = 19
MAX_Z = 120

BLOCK = 10000
ZSUB = 8
ZL = BLOCK // ZSUB


def _body(z_ref, gm_ref, feat_ref, table_ref, out_ref):
    out_ref[:, :D_FEAT] = feat_ref[...]

    z = z_ref[0, 0, :][:, None]
    gm = gm_ref[...]

    gcols = jax.lax.broadcasted_iota(jnp.int32, (MAX_Z, NUM_GROUPS), 1)
    ohg = (gm == gcols).astype(jnp.float32)
    fused = jnp.dot(ohg, table_ref[...],
                    preferred_element_type=jnp.float32)

    zcols = jax.lax.broadcasted_iota(jnp.int32, (BLOCK, MAX_Z), 1)
    ohz = (z == zcols).astype(jnp.float32)
    out_ref[:, D_FEAT:] = jnp.dot(ohz, fused,
                                  preferred_element_type=jnp.float32)


@functools.partial(jax.jit, static_argnames=())
def kernel(per_atom_property_tensor, atomic_numbers, table, group_map):
    n = per_atom_property_tensor.shape[0]
    grid = n // BLOCK
    z2d = atomic_numbers.astype(jnp.int32).reshape(grid, 1, BLOCK)
    gm2d = group_map.astype(jnp.int32).reshape(MAX_Z, 1)
    return pl.pallas_call(
        _body,
        grid=(grid,),
        in_specs=[
            pl.BlockSpec((1, 1, BLOCK), lambda i: (i, 0, 0)),
            pl.BlockSpec((MAX_Z, 1), lambda i: (0, 0)),
            pl.BlockSpec((BLOCK, D_FEAT), lambda i: (i, 0)),
            pl.BlockSpec((NUM_GROUPS, EMBED_DIM), lambda i: (0, 0)),
        ],
        out_specs=pl.BlockSpec((BLOCK, D_FEAT + EMBED_DIM), lambda i: (i, 0)),
        out_shape=jax.ShapeDtypeStruct((n, D_FEAT + EMBED_DIM), jnp.float32),
    )(z2d, gm2d, per_atom_property_tensor, table)


# speedup vs baseline: 33.0392x; 4.4974x over previous
import jax
import jax.numpy as jnp
from jax.experimental import pallas as pl

D_FEAT = 64
EMBED_DIM = 32
D_OUT = D_FEAT + EMBED_DIM
NUM_GROUPS = 19
MAX_Z = 120

BLOCK = 8192


def _body(z_ref, gm_ref, featT_ref, tableT_ref, outT_ref):
    outT_ref[:D_FEAT, :] = featT_ref[...]

    z = z_ref[...]
    gm = gm_ref[...]

    grows = jax.lax.broadcasted_iota(jnp.int32, (NUM_GROUPS, MAX_Z), 0)
    ohgT = (gm == grows).astype(jnp.float32)
    fusedT = jnp.dot(tableT_ref[...], ohgT,
                     preferred_element_type=jnp.float32)

    zrows = jax.lax.broadcasted_iota(jnp.int32, (MAX_Z, BLOCK), 0)
    ohzT = (z == zrows).astype(jnp.float32)
    outT_ref[D_FEAT:, :] = jnp.dot(fusedT, ohzT,
                                   preferred_element_type=jnp.float32)


def kernel(per_atom_property_tensor, atomic_numbers, table, group_map):
    n = per_atom_property_tensor.shape[0]
    grid = pl.cdiv(n, BLOCK)
    featT = per_atom_property_tensor.T
    tableT = table.T
    z2 = atomic_numbers.astype(jnp.int32).reshape(1, n)
    gm2 = group_map.astype(jnp.int32).reshape(1, MAX_Z)
    outT = pl.pallas_call(
        _body,
        grid=(grid,),
        in_specs=[
            pl.BlockSpec((1, BLOCK), lambda i: (0, i)),
            pl.BlockSpec((1, MAX_Z), lambda i: (0, 0)),
            pl.BlockSpec((D_FEAT, BLOCK), lambda i: (0, i)),
            pl.BlockSpec((EMBED_DIM, NUM_GROUPS), lambda i: (0, 0)),
        ],
        out_specs=pl.BlockSpec((D_OUT, BLOCK), lambda i: (0, i)),
        out_shape=jax.ShapeDtypeStruct((D_OUT, n), jnp.float32),
    )(z2, gm2, featT, tableT)
    return outT.T


# speedup vs baseline: 35.9773x; 1.0889x over previous
import jax
import jax.numpy as jnp
from jax.experimental import pallas as pl

D_FEAT = 64
EMBED_DIM = 32
D_OUT = D_FEAT + EMBED_DIM
NUM_GROUPS = 19
MAX_Z = 120

BLOCK = 16384


def _body(z_ref, gm_ref, featT_ref, tableT_ref, outT_ref):
    outT_ref[:D_FEAT, :] = featT_ref[...]

    z = z_ref[...]
    gm = gm_ref[...]

    grows = jax.lax.broadcasted_iota(jnp.int32, (NUM_GROUPS, MAX_Z), 0)
    ohgT = (gm == grows).astype(jnp.float32)
    fusedT = jnp.dot(tableT_ref[...], ohgT,
                     preferred_element_type=jnp.float32)

    zrows = jax.lax.broadcasted_iota(jnp.int32, (MAX_Z, BLOCK), 0)
    ohzT = (z == zrows).astype(jnp.float32)
    outT_ref[D_FEAT:, :] = jnp.dot(fusedT, ohzT,
                                   preferred_element_type=jnp.float32)


def kernel(per_atom_property_tensor, atomic_numbers, table, group_map):
    n = per_atom_property_tensor.shape[0]
    grid = pl.cdiv(n, BLOCK)
    featT = per_atom_property_tensor.T
    tableT = table.T
    z2 = atomic_numbers.astype(jnp.int32).reshape(1, n)
    gm2 = group_map.astype(jnp.int32).reshape(1, MAX_Z)
    outT = pl.pallas_call(
        _body,
        grid=(grid,),
        in_specs=[
            pl.BlockSpec((1, BLOCK), lambda i: (0, i)),
            pl.BlockSpec((1, MAX_Z), lambda i: (0, 0)),
            pl.BlockSpec((D_FEAT, BLOCK), lambda i: (0, i)),
            pl.BlockSpec((EMBED_DIM, NUM_GROUPS), lambda i: (0, 0)),
        ],
        out_specs=pl.BlockSpec((D_OUT, BLOCK), lambda i: (0, i)),
        out_shape=jax.ShapeDtypeStruct((D_OUT, n), jnp.float32),
    )(z2, gm2, featT, tableT)
    return outT.T


# speedup vs baseline: 37.0858x; 1.0308x over previous
import jax
import jax.numpy as jnp
from jax.experimental import pallas as pl

D_FEAT = 64
EMBED_DIM = 32
D_OUT = D_FEAT + EMBED_DIM
NUM_GROUPS = 19
MAX_Z = 120

BLOCK = 32768


def _body(z_ref, gm_ref, featT_ref, tableT_ref, outT_ref):
    outT_ref[:D_FEAT, :] = featT_ref[...]

    z = z_ref[...]
    gm = gm_ref[...]

    grows = jax.lax.broadcasted_iota(jnp.int32, (NUM_GROUPS, MAX_Z), 0)
    ohgT = (gm == grows).astype(jnp.float32)
    fusedT = jnp.dot(tableT_ref[...], ohgT,
                     preferred_element_type=jnp.float32)

    zrows = jax.lax.broadcasted_iota(jnp.int32, (MAX_Z, BLOCK), 0)
    ohzT = (z == zrows).astype(jnp.float32)
    outT_ref[D_FEAT:, :] = jnp.dot(fusedT, ohzT,
                                   preferred_element_type=jnp.float32)


def kernel(per_atom_property_tensor, atomic_numbers, table, group_map):
    n = per_atom_property_tensor.shape[0]
    grid = pl.cdiv(n, BLOCK)
    featT = per_atom_property_tensor.T
    tableT = table.T
    z2 = atomic_numbers.astype(jnp.int32).reshape(1, n)
    gm2 = group_map.astype(jnp.int32).reshape(1, MAX_Z)
    outT = pl.pallas_call(
        _body,
        grid=(grid,),
        in_specs=[
            pl.BlockSpec((1, BLOCK), lambda i: (0, i)),
            pl.BlockSpec((1, MAX_Z), lambda i: (0, 0)),
            pl.BlockSpec((D_FEAT, BLOCK), lambda i: (0, i)),
            pl.BlockSpec((EMBED_DIM, NUM_GROUPS), lambda i: (0, 0)),
        ],
        out_specs=pl.BlockSpec((D_OUT, BLOCK), lambda i: (0, i)),
        out_shape=jax.ShapeDtypeStruct((D_OUT, n), jnp.float32),
    )(z2, gm2, featT, tableT)
    return outT.T


# speedup vs baseline: 40.3085x; 1.0869x over previous
import jax
import jax.numpy as jnp
from jax.experimental import pallas as pl

D_FEAT = 64
EMBED_DIM = 32
D_OUT = D_FEAT + EMBED_DIM
NUM_GROUPS = 19
MAX_Z = 120

BLOCK = 32768


def _body(z_ref, gm_ref, featT_ref, tableT_ref, outT_ref):
    outT_ref[:D_FEAT, :] = featT_ref[...]

    z = z_ref[...][None, :]
    gm = gm_ref[...]

    grows = jax.lax.broadcasted_iota(jnp.int32, (NUM_GROUPS, MAX_Z), 0)
    ohgT = (gm == grows).astype(jnp.float32)
    fusedT = jnp.dot(tableT_ref[...], ohgT,
                     preferred_element_type=jnp.float32)

    zrows = jax.lax.broadcasted_iota(jnp.int32, (MAX_Z, BLOCK), 0)
    ohzT = (z == zrows).astype(jnp.float32)
    outT_ref[D_FEAT:, :] = jnp.dot(fusedT, ohzT,
                                   preferred_element_type=jnp.float32)


def kernel(per_atom_property_tensor, atomic_numbers, table, group_map):
    n = per_atom_property_tensor.shape[0]
    grid = pl.cdiv(n, BLOCK)
    featT = per_atom_property_tensor.T
    tableT = table.T
    z1 = atomic_numbers.astype(jnp.int32)
    gm2 = group_map.astype(jnp.int32).reshape(1, MAX_Z)
    outT = pl.pallas_call(
        _body,
        grid=(grid,),
        in_specs=[
            pl.BlockSpec((BLOCK,), lambda i: (i,)),
            pl.BlockSpec((1, MAX_Z), lambda i: (0, 0)),
            pl.BlockSpec((D_FEAT, BLOCK), lambda i: (0, i)),
            pl.BlockSpec((EMBED_DIM, NUM_GROUPS), lambda i: (0, 0)),
        ],
        out_specs=pl.BlockSpec((D_OUT, BLOCK), lambda i: (0, i)),
        out_shape=jax.ShapeDtypeStruct((D_OUT, n), jnp.float32),
    )(z1, gm2, featT, tableT)
    return outT.T


# speedup vs baseline: 40.5285x; 1.0055x over previous
import jax
import jax.numpy as jnp
from jax.experimental import pallas as pl

D_FEAT = 64
EMBED_DIM = 32
D_OUT = D_FEAT + EMBED_DIM
NUM_GROUPS = 19
MAX_Z = 120

BLOCK = 36864
CHUNK = 6144


def _body(z_ref, gm_ref, featT_ref, tableT_ref, outT_ref):
    outT_ref[:D_FEAT, :] = featT_ref[...]

    z = z_ref[...][None, :]
    gm = gm_ref[...]

    grows = jax.lax.broadcasted_iota(jnp.int32, (NUM_GROUPS, MAX_Z), 0)
    ohgT = (gm == grows).astype(jnp.float32)
    fusedT = jnp.dot(tableT_ref[...], ohgT,
                     preferred_element_type=jnp.float32)

    zrows = jax.lax.broadcasted_iota(jnp.int32, (MAX_Z, CHUNK), 0)
    for c in range(BLOCK // CHUNK):
        zc = z[:, c * CHUNK:(c + 1) * CHUNK]
        ohzT = (zc == zrows).astype(jnp.float32)
        outT_ref[D_FEAT:, c * CHUNK:(c + 1) * CHUNK] = jnp.dot(
            fusedT, ohzT, preferred_element_type=jnp.float32)


def kernel(per_atom_property_tensor, atomic_numbers, table, group_map):
    n = per_atom_property_tensor.shape[0]
    grid = pl.cdiv(n, BLOCK)
    featT = per_atom_property_tensor.T
    tableT = table.T
    z1 = atomic_numbers.astype(jnp.int32)
    gm2 = group_map.astype(jnp.int32).reshape(1, MAX_Z)
    outT = pl.pallas_call(
        _body,
        grid=(grid,),
        in_specs=[
            pl.BlockSpec((BLOCK,), lambda i: (i,)),
            pl.BlockSpec((1, MAX_Z), lambda i: (0, 0)),
            pl.BlockSpec((D_FEAT, BLOCK), lambda i: (0, i)),
            pl.BlockSpec((EMBED_DIM, NUM_GROUPS), lambda i: (0, 0)),
        ],
        out_specs=pl.BlockSpec((D_OUT, BLOCK), lambda i: (0, i)),
        out_shape=jax.ShapeDtypeStruct((D_OUT, n), jnp.float32),
    )(z1, gm2, featT, tableT)
    return outT.T
